# baseline (speedup 1.0000x reference)
import functools

import jax
import jax.numpy as jnp
from jax import lax
from jax.experimental import pallas as pl
from jax.experimental.pallas import tpu as pltpu
from jax.experimental.pallas import tpu_sc as plsc

VOCAB = 30522
MAX_POS = 512
CAT = 1536
NEW_EMB = 768
B = 64
T = 512
LN_EPS = 1e-12

NTOK = B * T
NC, NS, L = 2, 16, 16
NW = NC * NS
TPW = NTOK // NW
CH = 64
NV = NEW_EMB // L

ROW_BLK = 512
NBLK = (VOCAB + ROW_BLK - 1) // ROW_BLK


def _wm_body(word_ref, m_ref, wm_ref):
    wm_ref[...] = jnp.dot(word_ref[...], m_ref[...],
                          preferred_element_type=jnp.float32)


def _ptt_body(padd_ref, m_ref, ptt_ref):
    ptt_ref[...] = jnp.dot(padd_ref[...], m_ref[...],
                           preferred_element_type=jnp.float32)


def _tc_precompute(word_table, padd, M):
    wm = pl.pallas_call(
        _wm_body,
        grid=(NBLK,),
        in_specs=[
            pl.BlockSpec((ROW_BLK, CAT), lambda i: (i, 0)),
            pl.BlockSpec((CAT, NEW_EMB), lambda i: (0, 0)),
        ],
        out_specs=pl.BlockSpec((ROW_BLK, NEW_EMB), lambda i: (i, 0)),
        out_shape=jax.ShapeDtypeStruct((VOCAB, NEW_EMB), jnp.float32),
    )(word_table, M)
    ptt = pl.pallas_call(
        _ptt_body,
        out_shape=jax.ShapeDtypeStruct((2 * MAX_POS, NEW_EMB), jnp.float32),
    )(padd, M)
    return wm, ptt


def _sc_body(wm_hbm, ptt_hbm, widx_hbm, pidx_hbm, lnw_hbm, lnb_hbm, out_hbm,
             widx_v, pidx_v, w_v, p_v, lnw_v, lnb_v, red_v, sem1, sem2):
    c = lax.axis_index("c")
    s = lax.axis_index("s")
    wid = s * NC + c
    base0 = wid * TPW

    pltpu.sync_copy(lnw_hbm, lnw_v)
    pltpu.sync_copy(lnb_hbm, lnb_v)

    def chunk(ci, carry):
        base = base0 + ci * CH
        pltpu.sync_copy(widx_hbm.at[pl.ds(base, CH)], widx_v)
        pltpu.sync_copy(pidx_hbm.at[pl.ds(base, CH)], pidx_v)
        cp1 = pltpu.async_copy(wm_hbm.at[widx_v], w_v, sem1)
        cp2 = pltpu.async_copy(ptt_hbm.at[pidx_v], p_v, sem2)
        cp1.wait()
        cp2.wait()

        iota = lax.iota(jnp.int32, L)

        def lane_sum(v):
            t = v
            for sh in (1, 2, 4, 8):
                red_v[...] = t
                t = t + plsc.load_gather(red_v, [jnp.bitwise_xor(iota, sh)])
            return t

        def tok(i, tcarry):
            acc_s = jnp.zeros((L,), jnp.float32)
            acc_q = jnp.zeros((L,), jnp.float32)
            for j in range(NV):
                sl = pl.ds(j * L, L)
                x = w_v[i, sl] + p_v[i, sl]
                w_v[i, sl] = x
                acc_s = acc_s + x
                acc_q = acc_q + x * x
            ssum = lane_sum(acc_s)
            qsum = lane_sum(acc_q)
            mean = ssum * (1.0 / NEW_EMB)
            var = qsum * (1.0 / NEW_EMB) - mean * mean + LN_EPS
            ib = plsc.bitcast(var, jnp.int32)
            y = plsc.bitcast(
                jnp.full((L,), 0x5F3759DF, jnp.int32) - (ib >> 1), jnp.float32)
            for _ in range(3):
                y = y * (1.5 - 0.5 * var * y * y)
            s1 = y
            s0 = -mean * y
            for j in range(NV):
                sl = pl.ds(j * L, L)
                xn = w_v[i, sl] * s1 + s0
                w_v[i, sl] = xn * lnw_v[sl] + lnb_v[sl]
            return tcarry

        lax.fori_loop(0, CH, tok, 0)
        pltpu.sync_copy(w_v, out_hbm.at[pl.ds(base, CH)])
        return carry

    lax.fori_loop(0, TPW // CH, chunk, 0)


_sc_gather_ln = functools.partial(
    pl.kernel,
    out_type=jax.ShapeDtypeStruct((NTOK, NEW_EMB), jnp.float32),
    mesh=plsc.VectorSubcoreMesh(
        core_axis_name="c", subcore_axis_name="s",
        num_cores=NC, num_subcores=NS),
    compiler_params=pltpu.CompilerParams(needs_layout_passes=False),
    scratch_types=[
        pltpu.VMEM((CH,), jnp.int32),
        pltpu.VMEM((CH,), jnp.int32),
        pltpu.VMEM((CH, NEW_EMB), jnp.float32),
        pltpu.VMEM((CH, NEW_EMB), jnp.float32),
        pltpu.VMEM((NEW_EMB,), jnp.float32),
        pltpu.VMEM((NEW_EMB,), jnp.float32),
        pltpu.VMEM((L,), jnp.float32),
        pltpu.SemaphoreType.DMA,
        pltpu.SemaphoreType.DMA,
    ],
)(_sc_body)


@jax.jit
def kernel(input_ids, token_type_ids, word_table, position_table,
           token_type_table, M, ln_weight, ln_bias):
    widx = input_ids.reshape(-1).astype(jnp.int32)
    pidx = (token_type_ids.astype(jnp.int32) * MAX_POS
            + jnp.arange(T, dtype=jnp.int32)[None, :]).reshape(-1)
    padd = (position_table[None, :, :]
            + token_type_table[:, None, :]).reshape(2 * MAX_POS, CAT)
    wm, ptt = _tc_precompute(word_table, padd, M)
    out = _sc_gather_ln(wm, ptt, widx, pidx, ln_weight, ln_bias)
    return out.reshape(B, T, NEW_EMB)
